# baseline (speedup 1.0000x reference)
import functools

import jax
import jax.numpy as jnp
from jax.experimental import pallas as pl
from jax.experimental.pallas import tpu as pltpu


def _gem_body(p_ref, x_ref, o_ref, *, hw, eps):
    p = p_ref[0, 0]
    xc = jnp.maximum(x_ref[...], eps)
    xp = jnp.exp(jnp.log(xc) * p)
    s = jnp.sum(xp, axis=0)
    m = s * (1.0 / hw)
    o_ref[0, 0] = jnp.exp(jnp.log(m) / p)


def kernel(x, p):
    N, C, H, W = x.shape
    HW = H * W
    xt = jnp.transpose(x, (2, 3, 0, 1)).reshape(HW, N, C)

    p_arr = jnp.asarray(p, jnp.float32).reshape(1, 1)

    BN = 8
    while N % BN:
        BN //= 2

    out = pl.pallas_call(
        functools.partial(_gem_body, hw=HW, eps=1e-6),
        out_shape=jax.ShapeDtypeStruct((1, 1, N, C), x.dtype),
        grid=(N // BN,),
        in_specs=[
            pl.BlockSpec(memory_space=pltpu.MemorySpace.SMEM),
            pl.BlockSpec((HW, BN, C), lambda i: (0, i, 0)),
        ],
        out_specs=pl.BlockSpec((1, 1, BN, C), lambda i: (0, 0, i, 0)),
        compiler_params=pltpu.CompilerParams(
            dimension_semantics=("parallel",),
            vmem_limit_bytes=48 * 1024 * 1024),
        cost_estimate=pl.CostEstimate(
            flops=6 * N * C * HW,
            transcendentals=2 * N * C * HW + 2 * N * C,
            bytes_accessed=N * C * HW * 4 + N * C * 4),
    )(p_arr, xt)

    return jnp.transpose(out, (2, 3, 0, 1))


# speedup vs baseline: 1.1508x; 1.1508x over previous
import functools

import jax
import jax.numpy as jnp
from jax.experimental import pallas as pl
from jax.experimental.pallas import tpu as pltpu


def _gem_body(p_ref, x_ref, o_ref, *, hw, eps):
    p = p_ref[0, 0]
    xc = jnp.maximum(x_ref[...], eps)
    xp = jnp.exp(jnp.log(xc) * p)
    s = jnp.sum(xp, axis=0)
    m = s * (1.0 / hw)
    o_ref[0, 0] = jnp.exp(jnp.log(m) / p)


def kernel(x, p):
    N, C, H, W = x.shape
    HW = H * W
    xt = jnp.transpose(x, (2, 3, 0, 1)).reshape(HW, N, C)

    p_arr = jnp.asarray(p, jnp.float32).reshape(1, 1)

    BN = 16
    while N % BN:
        BN //= 2

    out = pl.pallas_call(
        functools.partial(_gem_body, hw=HW, eps=1e-6),
        out_shape=jax.ShapeDtypeStruct((1, 1, N, C), x.dtype),
        grid=(N // BN,),
        in_specs=[
            pl.BlockSpec(memory_space=pltpu.MemorySpace.SMEM),
            pl.BlockSpec((HW, BN, C), lambda i: (0, i, 0)),
        ],
        out_specs=pl.BlockSpec((1, 1, BN, C), lambda i: (0, 0, i, 0)),
        compiler_params=pltpu.CompilerParams(
            dimension_semantics=("parallel",),
            vmem_limit_bytes=48 * 1024 * 1024),
        cost_estimate=pl.CostEstimate(
            flops=6 * N * C * HW,
            transcendentals=2 * N * C * HW + 2 * N * C,
            bytes_accessed=N * C * HW * 4 + N * C * 4),
    )(p_arr, xt)

    return jnp.transpose(out, (2, 3, 0, 1))


# speedup vs baseline: 1.2304x; 1.0691x over previous
import functools

import jax
import jax.numpy as jnp
from jax.experimental import pallas as pl
from jax.experimental.pallas import tpu as pltpu


def _gem_body(p_ref, x_ref, o_ref, *, hw, eps):
    p = p_ref[0, 0]
    xc = jnp.maximum(x_ref[...], eps)
    xp = jnp.exp(jnp.log(xc) * p)
    s = jnp.sum(xp, axis=0)
    m = s * (1.0 / hw)
    o_ref[0, 0] = jnp.exp(jnp.log(m) / p)


def kernel(x, p):
    N, C, H, W = x.shape
    HW = H * W
    xt = jnp.transpose(x, (2, 3, 0, 1)).reshape(HW, N, C)

    p_arr = jnp.asarray(p, jnp.float32).reshape(1, 1)

    BN = 32
    while N % BN:
        BN //= 2

    out = pl.pallas_call(
        functools.partial(_gem_body, hw=HW, eps=1e-6),
        out_shape=jax.ShapeDtypeStruct((1, 1, N, C), x.dtype),
        grid=(N // BN,),
        in_specs=[
            pl.BlockSpec(memory_space=pltpu.MemorySpace.SMEM),
            pl.BlockSpec((HW, BN, C), lambda i: (0, i, 0)),
        ],
        out_specs=pl.BlockSpec((1, 1, BN, C), lambda i: (0, 0, i, 0)),
        compiler_params=pltpu.CompilerParams(
            dimension_semantics=("parallel",),
            vmem_limit_bytes=48 * 1024 * 1024),
        cost_estimate=pl.CostEstimate(
            flops=6 * N * C * HW,
            transcendentals=2 * N * C * HW + 2 * N * C,
            bytes_accessed=N * C * HW * 4 + N * C * 4),
    )(p_arr, xt)

    return jnp.transpose(out, (2, 3, 0, 1))
